# baseline (speedup 1.0000x reference)
import functools

import jax
import jax.numpy as jnp
from jax import lax
from jax.experimental import pallas as pl
from jax.experimental.pallas import tpu as pltpu
from jax.experimental.pallas import tpu_sc as plsc

N = 10000
D = 128
NC = 2
NS = 16
NW = NC * NS
CH = 72

STAGE = 40

@functools.lru_cache(maxsize=1)
def _get_mesh():
    return plsc.VectorSubcoreMesh(
        core_axis_name="c", subcore_axis_name="s", num_cores=NC, num_subcores=NS
    )


def _zero_vmem(buf, nrows):
    zeros16 = jnp.zeros((16,), jnp.float32)

    def body(i, _):
        for j in range(D // 16):
            buf[i, pl.ds(j * 16, 16)] = zeros16
        return 0

    lax.fori_loop(0, nrows, body, 0)


def _sc_deg_body(dst_hbm, ew_hbm, deg_out, deg_sh, dstv, eww, zbuf, nchunks):
    c = lax.axis_index("c")
    s = lax.axis_index("s")
    wid = s * NC + c

    def zb(i, _):
        zbuf[pl.ds(i * 16, 16)] = jnp.zeros((16,), jnp.float32)
        return 0

    lax.fori_loop(0, 63, zb, 0)

    @pl.when(s < 10)
    def _():
        pltpu.sync_copy(zbuf.at[pl.ds(0, 1000)], deg_sh.at[pl.ds(s * 1000, 1000)])

    plsc.subcore_barrier()

    def chunk(i, _):
        base = pl.multiple_of(wid * (nchunks * CH) + i * CH, 8)
        pltpu.sync_copy(dst_hbm.at[pl.ds(base, CH)], dstv)
        pltpu.sync_copy(ew_hbm.at[pl.ds(base, CH)], eww)
        pltpu.sync_copy(eww, deg_sh.at[dstv], add=True)
        return 0

    lax.fori_loop(0, nchunks, chunk, 0)
    plsc.subcore_barrier()

    @pl.when(s < 10)
    def _():
        base = pl.multiple_of(c * N + s * 1000, 8)
        pltpu.sync_copy(deg_sh.at[pl.ds(s * 1000, 1000)], zbuf.at[pl.ds(0, 1000)])
        pltpu.sync_copy(zbuf.at[pl.ds(0, 1000)], deg_out.at[pl.ds(base, 1000)])


def _sc_deg(dstp, ewp, nchunks):
    kfn = pl.kernel(
        functools.partial(_sc_deg_body, nchunks=nchunks),
        out_type=jax.ShapeDtypeStruct((NC * N,), jnp.float32),
        mesh=_get_mesh(),
        scratch_types=[
            pltpu.VMEM_SHARED((N,), jnp.float32),
            pltpu.VMEM((CH,), jnp.int32),
            pltpu.VMEM((CH,), jnp.float32),
            pltpu.VMEM((1008,), jnp.float32),
        ],
    )
    return kfn(dstp, ewp)


NSLOT = 4


def _sc_agg_body(y_hbm, src_hbm, dst_hbm, ewx_hbm, acc_out,
                 acc_sh, wrow0, wrow1, wrow2, wrow3, srcv0, srcv1, srcv2,
                 srcv3, dstv0, dstv1, dstv2, dstv3, rows0, rows1, rows2,
                 rows3, zbuf, gsem0, gsem1, gsem2, gsem3, nchunks):
    c = lax.axis_index("c")
    s = lax.axis_index("s")
    wid = s * NC + c
    ebase = wid * (nchunks * CH)
    slots = [
        (wrow0, srcv0, dstv0, rows0, gsem0),
        (wrow1, srcv1, dstv1, rows1, gsem1),
        (wrow2, srcv2, dstv2, rows2, gsem2),
        (wrow3, srcv3, dstv3, rows3, gsem3),
    ]

    def idx_load(i, slot):
        wrow, srcv, dstv, _, _ = slot
        base = pl.multiple_of(ebase + i * CH, 8)
        wbase = pl.multiple_of((ebase + i * CH) * 16, 8)
        pltpu.sync_copy(src_hbm.at[pl.ds(base, CH)], srcv)
        pltpu.sync_copy(dst_hbm.at[pl.ds(base, CH)], dstv)
        pltpu.sync_copy(ewx_hbm.at[pl.ds(wbase, CH * 16)], wrow)

    def gather_start(slot):
        _, srcv, _, rows, gsem = slot
        pltpu.async_copy(y_hbm.at[srcv], rows, gsem)

    def gather_wait(slot):
        _, srcv, _, rows, gsem = slot
        pltpu.make_async_copy(y_hbm.at[srcv], rows, gsem).wait()

    def scale_scatter(slot):
        wrow, _, dstv, rows, _ = slot

        @plsc.parallel_loop(0, CH, unroll=4)
        def _(e):
            wv = wrow[pl.ds(pl.multiple_of(e * 16, 16), 16)]
            for j in range(D // 16):
                rows[e, pl.ds(j * 16, 16)] = rows[e, pl.ds(j * 16, 16)] * wv

        pltpu.sync_copy(rows, acc_sh.at[dstv], add=True)

    for k in range(NSLOT):
        idx_load(k, slots[k])
        gather_start(slots[k])

    _zero_vmem(zbuf, STAGE)

    @pl.when(s < 10)
    def _():
        for k in range(1000 // STAGE):
            r0 = pl.multiple_of(s * 1000 + k * STAGE, 8)
            pltpu.sync_copy(zbuf, acc_sh.at[pl.ds(r0, STAGE)])

    plsc.subcore_barrier()

    def trip(g, _):
        i = NSLOT * g
        for k in range(NSLOT):
            slot = slots[k]
            gather_wait(slot)
            scale_scatter(slot)

            @pl.when(i + k + NSLOT < nchunks)
            def _():
                idx_load(i + k + NSLOT, slot)
                gather_start(slot)

        return 0

    lax.fori_loop(0, nchunks // NSLOT, trip, 0)
    plsc.subcore_barrier()

    @pl.when(s < 10)
    def _():
        for k in range(1000 // STAGE):
            r0 = pl.multiple_of(s * 1000 + k * STAGE, 8)
            pltpu.sync_copy(acc_sh.at[pl.ds(r0, STAGE)], zbuf)
            pltpu.sync_copy(zbuf, acc_out.at[c, pl.ds(r0, STAGE)])


def _sc_agg(y, srcp, dstp, ewp, nchunks):
    kfn = pl.kernel(
        functools.partial(_sc_agg_body, nchunks=nchunks),
        out_type=jax.ShapeDtypeStruct((NC, N, D), jnp.float32),
        mesh=_get_mesh(),
        scratch_types=[
            pltpu.VMEM_SHARED((N, D), jnp.float32),
            pltpu.VMEM((CH * 16,), jnp.float32),
            pltpu.VMEM((CH * 16,), jnp.float32),
            pltpu.VMEM((CH * 16,), jnp.float32),
            pltpu.VMEM((CH * 16,), jnp.float32),
            pltpu.VMEM((CH,), jnp.int32),
            pltpu.VMEM((CH,), jnp.int32),
            pltpu.VMEM((CH,), jnp.int32),
            pltpu.VMEM((CH,), jnp.int32),
            pltpu.VMEM((CH,), jnp.int32),
            pltpu.VMEM((CH,), jnp.int32),
            pltpu.VMEM((CH,), jnp.int32),
            pltpu.VMEM((CH,), jnp.int32),
            pltpu.VMEM((CH, D), jnp.float32),
            pltpu.VMEM((CH, D), jnp.float32),
            pltpu.VMEM((CH, D), jnp.float32),
            pltpu.VMEM((CH, D), jnp.float32),
            pltpu.VMEM((STAGE, D), jnp.float32),
            pltpu.SemaphoreType.DMA,
            pltpu.SemaphoreType.DMA,
            pltpu.SemaphoreType.DMA,
            pltpu.SemaphoreType.DMA,
        ],
    )
    return kfn(y, srcp, dstp, ewp)


BLK = 1000


def _tc_pre_body(deg_ref, x_ref, w_ref, y_ref, dis_ref):
    deg = deg_ref[0] + deg_ref[1] + 1.0
    dis = lax.rsqrt(deg)
    dis_ref[...] = dis
    y_ref[...] = jnp.dot(x_ref[...], w_ref[...],
                         preferred_element_type=jnp.float32) * dis


def _tc_pre(deg2, x, w1):
    return pl.pallas_call(
        _tc_pre_body,
        grid=(N // BLK,),
        in_specs=[
            pl.BlockSpec((NC, BLK, 1), lambda i: (0, i, 0)),
            pl.BlockSpec((BLK, D), lambda i: (i, 0)),
            pl.BlockSpec((D, D), lambda i: (0, 0)),
        ],
        out_specs=[
            pl.BlockSpec((BLK, D), lambda i: (i, 0)),
            pl.BlockSpec((BLK, 1), lambda i: (i, 0)),
        ],
        out_shape=[
            jax.ShapeDtypeStruct((N, D), jnp.float32),
            jax.ShapeDtypeStruct((N, 1), jnp.float32),
        ],
    )(deg2, x, w1)


def _tc_mid_body(acc_ref, y_ref, dis_ref, b_ref, w_ref, o_ref):
    dis = dis_ref[...]
    h = (acc_ref[0] + acc_ref[1] + y_ref[...]) * dis + b_ref[...]
    h = jnp.maximum(h, 0.0)
    o_ref[...] = jnp.dot(h, w_ref[...], preferred_element_type=jnp.float32) * dis


def _tc_mid(acc2, y, dis, b, w_next):
    return pl.pallas_call(
        _tc_mid_body,
        grid=(N // BLK,),
        in_specs=[
            pl.BlockSpec((NC, BLK, D), lambda i: (0, i, 0)),
            pl.BlockSpec((BLK, D), lambda i: (i, 0)),
            pl.BlockSpec((BLK, 1), lambda i: (i, 0)),
            pl.BlockSpec((1, D), lambda i: (0, 0)),
            pl.BlockSpec((D, D), lambda i: (0, 0)),
        ],
        out_specs=pl.BlockSpec((BLK, D), lambda i: (i, 0)),
        out_shape=jax.ShapeDtypeStruct((N, D), jnp.float32),
    )(acc2, y, dis, b, w_next)


def _tc_post_body(acc_ref, y_ref, dis_ref, b_ref, o_ref):
    h = (acc_ref[0] + acc_ref[1] + y_ref[...]) * dis_ref[...] + b_ref[...]
    o_ref[...] = jnp.maximum(h, 0.0)


def _tc_post(acc2, y, dis, b):
    return pl.pallas_call(
        _tc_post_body,
        grid=(N // BLK,),
        in_specs=[
            pl.BlockSpec((NC, BLK, D), lambda i: (0, i, 0)),
            pl.BlockSpec((BLK, D), lambda i: (i, 0)),
            pl.BlockSpec((BLK, 1), lambda i: (i, 0)),
            pl.BlockSpec((1, D), lambda i: (0, 0)),
        ],
        out_specs=pl.BlockSpec((BLK, D), lambda i: (i, 0)),
        out_shape=jax.ShapeDtypeStruct((N, D), jnp.float32),
    )(acc2, y, dis, b)


def kernel(x, edge_index, edge_attr, W1, b1, W2, b2, W3, b3):
    E = edge_index.shape[1]
    nchunks = -(-E // (NW * CH))
    nchunks += (-nchunks) % NSLOT
    e_pad = NW * CH * nchunks
    pad = e_pad - E
    src = jnp.concatenate([edge_index[0], jnp.zeros((pad,), edge_index.dtype)])
    dst = jnp.concatenate([edge_index[1], jnp.zeros((pad,), edge_index.dtype)])
    ew = jnp.concatenate([edge_attr, jnp.zeros((pad,), edge_attr.dtype)])
    ewx = jnp.broadcast_to(ew[:, None], (e_pad, 16)).reshape(-1)

    deg2 = _sc_deg(dst, ew, nchunks)
    deg2 = deg2.reshape(NC, N, 1)

    b1r = b1.reshape(1, D)
    b2r = b2.reshape(1, D)
    b3r = b3.reshape(1, D)

    y1, dis = _tc_pre(deg2, x, W1)
    acc1 = _sc_agg(y1, src, dst, ewx, nchunks)
    y2 = _tc_mid(acc1, y1, dis, b1r, W2)
    acc2 = _sc_agg(y2, src, dst, ewx, nchunks)
    y3 = _tc_mid(acc2, y2, dis, b2r, W3)
    acc3 = _sc_agg(y3, src, dst, ewx, nchunks)
    return _tc_post(acc3, y3, dis, b3r)


# speedup vs baseline: 1.1022x; 1.1022x over previous
import functools

import jax
import jax.numpy as jnp
from jax import lax
from jax.experimental import pallas as pl
from jax.experimental.pallas import tpu as pltpu
from jax.experimental.pallas import tpu_sc as plsc

N = 10000
D = 128
NC = 2
NS = 16
NW = NC * NS
CH = 96

STAGE = 40

@functools.lru_cache(maxsize=1)
def _get_mesh():
    return plsc.VectorSubcoreMesh(
        core_axis_name="c", subcore_axis_name="s", num_cores=NC, num_subcores=NS
    )


def _zero_vmem(buf, nrows):
    zeros16 = jnp.zeros((16,), jnp.float32)

    def body(i, _):
        for j in range(D // 16):
            buf[i, pl.ds(j * 16, 16)] = zeros16
        return 0

    lax.fori_loop(0, nrows, body, 0)


def _sc_deg_body(dst_hbm, ew_hbm, deg_out, deg_sh, dstv0, dstv1, dstv2,
                 eww0, eww1, eww2, zbuf, dsem0, dsem1, dsem2, nchunks):
    c = lax.axis_index("c")
    s = lax.axis_index("s")
    wid = s * NC + c
    slots = [(dstv0, eww0, dsem0), (dstv1, eww1, dsem1), (dstv2, eww2, dsem2)]

    def dload(i, slot):
        dstv, eww, sem = slot
        base = pl.multiple_of(wid * (nchunks * CH) + i * CH, 8)
        pltpu.async_copy(dst_hbm.at[pl.ds(base, CH)], dstv, sem)
        pltpu.async_copy(ew_hbm.at[pl.ds(base, CH)], eww, sem)

    def dwait(slot):
        dstv, eww, sem = slot
        pltpu.make_async_copy(dst_hbm.at[pl.ds(0, CH)], dstv, sem).wait()
        pltpu.make_async_copy(ew_hbm.at[pl.ds(0, CH)], eww, sem).wait()

    for k in range(NSLOT):
        dload(k, slots[k])

    def zb(i, _):
        zbuf[pl.ds(i * 16, 16)] = jnp.zeros((16,), jnp.float32)
        return 0

    lax.fori_loop(0, 63, zb, 0)

    @pl.when(s < 10)
    def _():
        pltpu.sync_copy(zbuf.at[pl.ds(0, 1000)], deg_sh.at[pl.ds(s * 1000, 1000)])

    plsc.subcore_barrier()

    def trip(g, _):
        i = NSLOT * g
        for k in range(NSLOT):
            slot = slots[k]
            dwait(slot)
            dstv, eww, _ = slot
            pltpu.sync_copy(eww, deg_sh.at[dstv], add=True)

            @pl.when(i + k + NSLOT < nchunks)
            def _():
                dload(i + k + NSLOT, slot)

        return 0

    lax.fori_loop(0, nchunks // NSLOT, trip, 0)
    plsc.subcore_barrier()

    @pl.when(s < 10)
    def _():
        base = pl.multiple_of(c * N + s * 1000, 8)
        pltpu.sync_copy(deg_sh.at[pl.ds(s * 1000, 1000)], zbuf.at[pl.ds(0, 1000)])
        pltpu.sync_copy(zbuf.at[pl.ds(0, 1000)], deg_out.at[pl.ds(base, 1000)])


def _sc_deg(dstp, ewp, nchunks):
    kfn = pl.kernel(
        functools.partial(_sc_deg_body, nchunks=nchunks),
        out_type=jax.ShapeDtypeStruct((NC * N,), jnp.float32),
        mesh=_get_mesh(),
        scratch_types=[
            pltpu.VMEM_SHARED((N,), jnp.float32),
            pltpu.VMEM((CH,), jnp.int32),
            pltpu.VMEM((CH,), jnp.int32),
            pltpu.VMEM((CH,), jnp.int32),
            pltpu.VMEM((CH,), jnp.float32),
            pltpu.VMEM((CH,), jnp.float32),
            pltpu.VMEM((CH,), jnp.float32),
            pltpu.VMEM((1008,), jnp.float32),
            pltpu.SemaphoreType.DMA,
            pltpu.SemaphoreType.DMA,
            pltpu.SemaphoreType.DMA,
        ],
    )
    return kfn(dstp, ewp)


NSLOT = 3


def _sc_agg_body(y_hbm, src_hbm, dst_hbm, ewx_hbm, acc_out,
                 acc_sh, wrow0, wrow1, wrow2, srcv0, srcv1, srcv2,
                 dstv0, dstv1, dstv2, rows0, rows1, rows2, zbuf,
                 gsem0, gsem1, gsem2, nchunks):
    c = lax.axis_index("c")
    s = lax.axis_index("s")
    wid = s * NC + c
    ebase = wid * (nchunks * CH)
    slots = [
        (wrow0, srcv0, dstv0, rows0, gsem0),
        (wrow1, srcv1, dstv1, rows1, gsem1),
        (wrow2, srcv2, dstv2, rows2, gsem2),
    ]

    def idx_load(i, slot):
        wrow, srcv, dstv, _, _ = slot
        base = pl.multiple_of(ebase + i * CH, 8)
        wbase = pl.multiple_of((ebase + i * CH) * 16, 8)
        pltpu.sync_copy(src_hbm.at[pl.ds(base, CH)], srcv)
        pltpu.sync_copy(dst_hbm.at[pl.ds(base, CH)], dstv)
        pltpu.sync_copy(ewx_hbm.at[pl.ds(wbase, CH * 16)], wrow)

    def gather_start(slot):
        _, srcv, _, rows, gsem = slot
        pltpu.async_copy(y_hbm.at[srcv], rows, gsem)

    def gather_wait(slot):
        _, srcv, _, rows, gsem = slot
        pltpu.make_async_copy(y_hbm.at[srcv], rows, gsem).wait()

    def scale_scatter(slot):
        wrow, _, dstv, rows, _ = slot

        @plsc.parallel_loop(0, CH, unroll=4)
        def _(e):
            wv = wrow[pl.ds(pl.multiple_of(e * 16, 16), 16)]
            for j in range(D // 16):
                rows[e, pl.ds(j * 16, 16)] = rows[e, pl.ds(j * 16, 16)] * wv

        pltpu.sync_copy(rows, acc_sh.at[dstv], add=True)

    for k in range(NSLOT):
        idx_load(k, slots[k])
        gather_start(slots[k])

    _zero_vmem(zbuf, STAGE)

    @pl.when(s < 10)
    def _():
        for k in range(1000 // STAGE):
            r0 = pl.multiple_of(s * 1000 + k * STAGE, 8)
            pltpu.sync_copy(zbuf, acc_sh.at[pl.ds(r0, STAGE)])

    plsc.subcore_barrier()

    def trip(g, _):
        i = NSLOT * g
        for k in range(NSLOT):
            slot = slots[k]
            gather_wait(slot)
            scale_scatter(slot)

            @pl.when(i + k + NSLOT < nchunks)
            def _():
                idx_load(i + k + NSLOT, slot)
                gather_start(slot)

        return 0

    lax.fori_loop(0, nchunks // NSLOT, trip, 0)
    plsc.subcore_barrier()

    @pl.when(s < 10)
    def _():
        for k in range(1000 // STAGE):
            r0 = pl.multiple_of(s * 1000 + k * STAGE, 8)
            pltpu.sync_copy(acc_sh.at[pl.ds(r0, STAGE)], zbuf)
            pltpu.sync_copy(zbuf, acc_out.at[c, pl.ds(r0, STAGE)])


def _sc_agg(y, srcp, dstp, ewp, nchunks):
    kfn = pl.kernel(
        functools.partial(_sc_agg_body, nchunks=nchunks),
        out_type=jax.ShapeDtypeStruct((NC, N, D), jnp.float32),
        mesh=_get_mesh(),
        scratch_types=[
            pltpu.VMEM_SHARED((N, D), jnp.float32),
            pltpu.VMEM((CH * 16,), jnp.float32),
            pltpu.VMEM((CH * 16,), jnp.float32),
            pltpu.VMEM((CH * 16,), jnp.float32),
            pltpu.VMEM((CH,), jnp.int32),
            pltpu.VMEM((CH,), jnp.int32),
            pltpu.VMEM((CH,), jnp.int32),
            pltpu.VMEM((CH,), jnp.int32),
            pltpu.VMEM((CH,), jnp.int32),
            pltpu.VMEM((CH,), jnp.int32),
            pltpu.VMEM((CH, D), jnp.float32),
            pltpu.VMEM((CH, D), jnp.float32),
            pltpu.VMEM((CH, D), jnp.float32),
            pltpu.VMEM((STAGE, D), jnp.float32),
            pltpu.SemaphoreType.DMA,
            pltpu.SemaphoreType.DMA,
            pltpu.SemaphoreType.DMA,
        ],
    )
    return kfn(y, srcp, dstp, ewp)


BLK = 1000


def _tc_z_body(x_ref, w_ref, z_ref):
    z_ref[...] = jnp.dot(x_ref[...], w_ref[...],
                         preferred_element_type=jnp.float32)


def _tc_z(x, w1):
    return pl.pallas_call(
        _tc_z_body,
        grid=(N // BLK,),
        in_specs=[
            pl.BlockSpec((BLK, D), lambda i: (i, 0)),
            pl.BlockSpec((D, D), lambda i: (0, 0)),
        ],
        out_specs=pl.BlockSpec((BLK, D), lambda i: (i, 0)),
        out_shape=jax.ShapeDtypeStruct((N, D), jnp.float32),
    )(x, w1)


def _tc_scale_body(deg_ref, z_ref, y_ref, dis_ref):
    deg = deg_ref[0] + deg_ref[1] + 1.0
    dis = lax.rsqrt(deg)
    dis_ref[...] = dis
    y_ref[...] = z_ref[...] * dis


def _tc_scale(deg2, z):
    return pl.pallas_call(
        _tc_scale_body,
        grid=(N // BLK,),
        in_specs=[
            pl.BlockSpec((NC, BLK, 1), lambda i: (0, i, 0)),
            pl.BlockSpec((BLK, D), lambda i: (i, 0)),
        ],
        out_specs=[
            pl.BlockSpec((BLK, D), lambda i: (i, 0)),
            pl.BlockSpec((BLK, 1), lambda i: (i, 0)),
        ],
        out_shape=[
            jax.ShapeDtypeStruct((N, D), jnp.float32),
            jax.ShapeDtypeStruct((N, 1), jnp.float32),
        ],
    )(deg2, z)


def _tc_mid_body(acc_ref, y_ref, dis_ref, b_ref, w_ref, o_ref):
    dis = dis_ref[...]
    h = (acc_ref[0] + acc_ref[1] + y_ref[...]) * dis + b_ref[...]
    h = jnp.maximum(h, 0.0)
    o_ref[...] = jnp.dot(h, w_ref[...], preferred_element_type=jnp.float32) * dis


def _tc_mid(acc2, y, dis, b, w_next):
    return pl.pallas_call(
        _tc_mid_body,
        grid=(N // BLK,),
        in_specs=[
            pl.BlockSpec((NC, BLK, D), lambda i: (0, i, 0)),
            pl.BlockSpec((BLK, D), lambda i: (i, 0)),
            pl.BlockSpec((BLK, 1), lambda i: (i, 0)),
            pl.BlockSpec((1, D), lambda i: (0, 0)),
            pl.BlockSpec((D, D), lambda i: (0, 0)),
        ],
        out_specs=pl.BlockSpec((BLK, D), lambda i: (i, 0)),
        out_shape=jax.ShapeDtypeStruct((N, D), jnp.float32),
    )(acc2, y, dis, b, w_next)


def _tc_post_body(acc_ref, y_ref, dis_ref, b_ref, o_ref):
    h = (acc_ref[0] + acc_ref[1] + y_ref[...]) * dis_ref[...] + b_ref[...]
    o_ref[...] = jnp.maximum(h, 0.0)


def _tc_post(acc2, y, dis, b):
    return pl.pallas_call(
        _tc_post_body,
        grid=(N // BLK,),
        in_specs=[
            pl.BlockSpec((NC, BLK, D), lambda i: (0, i, 0)),
            pl.BlockSpec((BLK, D), lambda i: (i, 0)),
            pl.BlockSpec((BLK, 1), lambda i: (i, 0)),
            pl.BlockSpec((1, D), lambda i: (0, 0)),
        ],
        out_specs=pl.BlockSpec((BLK, D), lambda i: (i, 0)),
        out_shape=jax.ShapeDtypeStruct((N, D), jnp.float32),
    )(acc2, y, dis, b)


def kernel(x, edge_index, edge_attr, W1, b1, W2, b2, W3, b3):
    E = edge_index.shape[1]
    nchunks = -(-E // (NW * CH))
    nchunks += (-nchunks) % NSLOT
    e_pad = NW * CH * nchunks
    pad = e_pad - E
    src = jnp.concatenate([edge_index[0], jnp.zeros((pad,), edge_index.dtype)])
    dst = jnp.concatenate([edge_index[1], jnp.zeros((pad,), edge_index.dtype)])
    ew = jnp.concatenate([edge_attr, jnp.zeros((pad,), edge_attr.dtype)])
    ewx = jnp.broadcast_to(ew[:, None], (e_pad, 16)).reshape(-1)

    z1 = _tc_z(x, W1)
    deg2 = _sc_deg(dst, ew, nchunks)
    deg2 = deg2.reshape(NC, N, 1)

    b1r = b1.reshape(1, D)
    b2r = b2.reshape(1, D)
    b3r = b3.reshape(1, D)

    y1, dis = _tc_scale(deg2, z1)
    acc1 = _sc_agg(y1, src, dst, ewx, nchunks)
    y2 = _tc_mid(acc1, y1, dis, b1r, W2)
    acc2 = _sc_agg(y2, src, dst, ewx, nchunks)
    y3 = _tc_mid(acc2, y2, dis, b2r, W3)
    acc3 = _sc_agg(y3, src, dst, ewx, nchunks)
    return _tc_post(acc3, y3, dis, b3r)
